# baseline (speedup 1.0000x reference)
import functools

import jax
import jax.numpy as jnp
from jax import lax
from jax.experimental import pallas as pl
from jax.experimental.pallas import tpu as pltpu
from jax.experimental.pallas import tpu_sc as plsc

N = 10000
E = 320000
D = 128

NC = 2
NS = 16
LANES = 16

CHUNK = 128
HALF = 5120
N_PAD = 2 * HALF
RPH = HALF // NS
CAPC = 2560
CAPE = CAPC * CHUNK

_mesh = plsc.VectorSubcoreMesh(
    core_axis_name="c", subcore_axis_name="s", num_cores=NC, num_subcores=NS)


@functools.partial(
    pl.kernel,
    out_type=jax.ShapeDtypeStruct((N_PAD, D), jnp.float32),
    mesh=_mesh,
    scratch_types=[
        pltpu.VMEM((2, CHUNK, D), jnp.float32),
        pltpu.VMEM((2, 2, CHUNK), jnp.int32),
        pltpu.VMEM((2, LANES, CHUNK), jnp.float32),
        pltpu.VMEM((8, 128), jnp.int32),
        pltpu.VMEM_SHARED((HALF, D), jnp.float32),
        pltpu.SemaphoreType.DMA, pltpu.SemaphoreType.DMA,
    ],
)
def _spmm_sc(h_hbm, zeros_hbm, idx_hbm, w_hbm, cnt_hbm, out_hbm,
             rows_v, idx_v, w_v, cnt_v, acc_sh, sg0, sg1):
    c = lax.axis_index("c")
    s = lax.axis_index("s")
    sg = (sg0, sg1)

    pltpu.sync_copy(cnt_hbm.at[c], cnt_v)
    cpw = cnt_v[0, pl.ds(0, LANES)][0]

    pltpu.sync_copy(zeros_hbm.at[pl.ds(s * RPH, RPH)],
                    acc_sh.at[pl.ds(s * RPH, RPH)])
    plsc.subcore_barrier()

    def meta_copy_sync(slot, t):
        g = NS * t + s
        pltpu.sync_copy(idx_hbm.at[c, g], idx_v.at[slot])
        pltpu.sync_copy(w_hbm.at[c, g], w_v.at[slot])

    def gather_start(slot):
        pltpu.async_copy(h_hbm.at[idx_v.at[slot, 0]], rows_v.at[slot],
                         sg[slot])

    def gather_wait(slot):
        pltpu.make_async_copy(h_hbm.at[idx_v.at[slot, 0]], rows_v.at[slot],
                              sg[slot]).wait()

    meta_copy_sync(0, 0)
    gather_start(0)

    def step(j, b):
        nb = 1 - b
        @pl.when(j + 1 < cpw)
        def _():
            meta_copy_sync(nb, j + 1)
            gather_start(nb)

        @pl.when(j < cpw)
        def _():
            gather_wait(b)

            def edge_body(i, carry):
                wv = w_v[b, i // 8, pl.ds((i % 8) * LANES, LANES)]
                for jj in range(D // LANES):
                    sl = (b, i, pl.ds(jj * LANES, LANES))
                    rows_v[sl] = rows_v[sl] * wv
                return carry
            lax.fori_loop(0, CHUNK, edge_body, 0, unroll=4)

            pltpu.sync_copy(rows_v.at[b], acc_sh.at[idx_v.at[b, 1]],
                            add=True)

    def loop_body(jj, carry):
        step(2 * jj, 0)
        step(2 * jj + 1, 1)
        return carry
    lax.fori_loop(0, CAPC // NS // 2, loop_body, 0)

    plsc.subcore_barrier()

    pltpu.sync_copy(acc_sh.at[pl.ds(s * RPH, RPH)],
                    out_hbm.at[pl.ds(c * HALF + s * RPH, RPH)])


_BN = 1000


def _linear_tc(h, W, b2):
    def body(h_ref, w_ref, b_ref, o_ref):
        acc = lax.dot_general(h_ref[...], w_ref[...],
                              (((1,), (1,)), ((), ())),
                              preferred_element_type=jnp.float32)
        o_ref[...] = acc + b_ref[...]
    return pl.pallas_call(
        body,
        grid=(N // _BN,),
        in_specs=[
            pl.BlockSpec((_BN, D), lambda i: (i, 0)),
            pl.BlockSpec((D, D), lambda i: (0, 0)),
            pl.BlockSpec((1, D), lambda i: (0, 0)),
        ],
        out_specs=pl.BlockSpec((_BN, D), lambda i: (i, 0)),
        out_shape=jax.ShapeDtypeStruct((N, D), jnp.float32),
    )(h, W, b2)


def kernel(x, edge_index, edge_weight, W, b, k):
    row = edge_index[0]
    col = edge_index[1]

    side = row >= HALF
    sidx = jnp.cumsum(side.astype(jnp.int32))
    n1 = sidx[-1]
    n0 = E - n1
    ar = jnp.arange(E, dtype=jnp.int32)
    pos = jnp.where(side, sidx - 1, ar - sidx)
    tgt = jnp.where(side, CAPE + pos, pos)
    colp = jnp.zeros((2 * CAPE,), jnp.int32).at[tgt].set(col)
    rowp = jnp.zeros((2 * CAPE,), jnp.int32).at[tgt].set(
        row - side.astype(jnp.int32) * HALF)
    wp = jnp.zeros((2 * CAPE,), jnp.float32).at[tgt].set(edge_weight)

    idx = jnp.concatenate(
        [colp.reshape(2, CAPC, 1, CHUNK), rowp.reshape(2, CAPC, 1, CHUNK)],
        axis=2)
    wexp = jnp.broadcast_to(
        wp.reshape(2, CAPC, CHUNK, 1),
        (2, CAPC, CHUNK, LANES)).reshape(2, CAPC, LANES, CHUNK)

    def _cpw(n):
        chunks = (n + CHUNK - 1) // CHUNK
        return jnp.maximum(4 * ((chunks + 4 * NS - 1) // (4 * NS)), 4)
    cnts = jnp.broadcast_to(
        jnp.stack([_cpw(n0), _cpw(n1)]).astype(jnp.int32)[:, None, None],
        (2, 8, 128))

    zeros = jnp.zeros((HALF, D), jnp.float32)
    b2 = b.reshape(1, D)
    x_pad = jnp.pad(x, ((0, N_PAD - N), (0, 0)))

    def it_body(_, h):
        return _spmm_sc(h, zeros, idx, wexp, cnts)

    h = lax.fori_loop(0, k, it_body, x_pad)
    return _linear_tc(h[:N], W, b2)


# speedup vs baseline: 1.0006x; 1.0006x over previous
import functools

import jax
import jax.numpy as jnp
from jax import lax
from jax.experimental import pallas as pl
from jax.experimental.pallas import tpu as pltpu
from jax.experimental.pallas import tpu_sc as plsc

N = 10000
E = 320000
D = 128

NC = 2
NS = 16
LANES = 16

CHUNK = 128
HALF = 5120
N_PAD = 2 * HALF
RPH = HALF // NS
CAPC = 2560
CAPE = CAPC * CHUNK

_mesh = plsc.VectorSubcoreMesh(
    core_axis_name="c", subcore_axis_name="s", num_cores=NC, num_subcores=NS)


@functools.partial(
    pl.kernel,
    out_type=jax.ShapeDtypeStruct((N_PAD, D), jnp.float32),
    mesh=_mesh,
    scratch_types=[
        pltpu.VMEM((2, CHUNK, D), jnp.float32),
        pltpu.VMEM((2, 2, CHUNK), jnp.int32),
        pltpu.VMEM((2, LANES, CHUNK), jnp.float32),
        pltpu.VMEM((8, 128), jnp.int32),
        pltpu.VMEM_SHARED((HALF, D), jnp.float32),
        pltpu.SemaphoreType.DMA, pltpu.SemaphoreType.DMA,
    ],
)
def _spmm_sc(h_hbm, zeros_hbm, idx_hbm, w_hbm, cnt_hbm, out_hbm,
             rows_v, idx_v, w_v, cnt_v, acc_sh, sg0, sg1):
    c = lax.axis_index("c")
    s = lax.axis_index("s")
    sg = (sg0, sg1)

    pltpu.sync_copy(cnt_hbm.at[c], cnt_v)
    cpw = cnt_v[0, pl.ds(0, LANES)][0]

    pltpu.sync_copy(zeros_hbm.at[pl.ds(s * RPH, RPH)],
                    acc_sh.at[pl.ds(s * RPH, RPH)])
    plsc.subcore_barrier()

    def meta_copy_sync(slot, t):
        g = NS * t + s
        pltpu.sync_copy(idx_hbm.at[c, g], idx_v.at[slot])
        pltpu.sync_copy(w_hbm.at[c, g], w_v.at[slot])

    def gather_start(slot):
        pltpu.async_copy(h_hbm.at[idx_v.at[slot, 0]], rows_v.at[slot],
                         sg[slot])

    def gather_wait(slot):
        pltpu.make_async_copy(h_hbm.at[idx_v.at[slot, 0]], rows_v.at[slot],
                              sg[slot]).wait()

    meta_copy_sync(0, 0)
    gather_start(0)

    def step(j, b, prefetch):
        nb = 1 - b
        if prefetch:
            meta_copy_sync(nb, j + 1)
            gather_start(nb)

        gather_wait(b)

        def edge_body(i, carry):
            wv = w_v[b, i // 8, pl.ds((i % 8) * LANES, LANES)]
            for jj in range(D // LANES):
                sl = (b, i, pl.ds(jj * LANES, LANES))
                rows_v[sl] = rows_v[sl] * wv
            return carry
        lax.fori_loop(0, CHUNK, edge_body, 0, unroll=4)

        pltpu.sync_copy(rows_v.at[b], acc_sh.at[idx_v.at[b, 1]], add=True)

    def loop_body(jj, carry):
        step(2 * jj, 0, True)
        step(2 * jj + 1, 1, True)
        return carry
    lax.fori_loop(0, (cpw - 2) // 2, loop_body, 0)
    step(cpw - 2, 0, True)
    step(cpw - 1, 1, False)

    plsc.subcore_barrier()

    pltpu.sync_copy(acc_sh.at[pl.ds(s * RPH, RPH)],
                    out_hbm.at[pl.ds(c * HALF + s * RPH, RPH)])


_BN = 1000


def _linear_tc(h, W, b2):
    def body(h_ref, w_ref, b_ref, o_ref):
        acc = lax.dot_general(h_ref[...], w_ref[...],
                              (((1,), (1,)), ((), ())),
                              preferred_element_type=jnp.float32)
        o_ref[...] = acc + b_ref[...]
    return pl.pallas_call(
        body,
        grid=(N // _BN,),
        in_specs=[
            pl.BlockSpec((_BN, D), lambda i: (i, 0)),
            pl.BlockSpec((D, D), lambda i: (0, 0)),
            pl.BlockSpec((1, D), lambda i: (0, 0)),
        ],
        out_specs=pl.BlockSpec((_BN, D), lambda i: (i, 0)),
        out_shape=jax.ShapeDtypeStruct((N, D), jnp.float32),
    )(h, W, b2)


def kernel(x, edge_index, edge_weight, W, b, k):
    row = edge_index[0]
    col = edge_index[1]

    side = row >= HALF
    sidx = jnp.cumsum(side.astype(jnp.int32))
    n1 = sidx[-1]
    n0 = E - n1
    ar = jnp.arange(E, dtype=jnp.int32)
    pos = jnp.where(side, sidx - 1, ar - sidx)
    tgt = jnp.where(side, CAPE + pos, pos)
    colp = jnp.zeros((2 * CAPE,), jnp.int32).at[tgt].set(col)
    rowp = jnp.zeros((2 * CAPE,), jnp.int32).at[tgt].set(
        row - side.astype(jnp.int32) * HALF)
    wp = jnp.zeros((2 * CAPE,), jnp.float32).at[tgt].set(edge_weight)

    idx = jnp.concatenate(
        [colp.reshape(2, CAPC, 1, CHUNK), rowp.reshape(2, CAPC, 1, CHUNK)],
        axis=2)
    wexp = jnp.broadcast_to(
        wp.reshape(2, CAPC, CHUNK, 1),
        (2, CAPC, CHUNK, LANES)).reshape(2, CAPC, LANES, CHUNK)

    def _cpw(n):
        chunks = (n + CHUNK - 1) // CHUNK
        return jnp.maximum(4 * ((chunks + 4 * NS - 1) // (4 * NS)), 4)
    cnts = jnp.broadcast_to(
        jnp.stack([_cpw(n0), _cpw(n1)]).astype(jnp.int32)[:, None, None],
        (2, 8, 128))

    zeros = jnp.zeros((HALF, D), jnp.float32)
    b2 = b.reshape(1, D)
    x_pad = jnp.pad(x, ((0, N_PAD - N), (0, 0)))

    def it_body(_, h):
        return _spmm_sc(h, zeros, idx, wexp, cnts)

    h = lax.fori_loop(0, k, it_body, x_pad)
    return _linear_tc(h[:N], W, b2)


# speedup vs baseline: 1.0681x; 1.0674x over previous
import functools

import jax
import jax.numpy as jnp
from jax import lax
from jax.experimental import pallas as pl
from jax.experimental.pallas import tpu as pltpu
from jax.experimental.pallas import tpu_sc as plsc

N = 10000
E = 320000
D = 128

NC = 2
NS = 16
LANES = 16

CHUNK = 128
HALF = 5120
N_PAD = 2 * HALF
RPH = HALF // NS
CAPC = 2560
CAPE = CAPC * CHUNK

_mesh = plsc.VectorSubcoreMesh(
    core_axis_name="c", subcore_axis_name="s", num_cores=NC, num_subcores=NS)


@functools.partial(
    pl.kernel,
    out_type=jax.ShapeDtypeStruct((N_PAD, D), jnp.float32),
    mesh=_mesh,
    scratch_types=[
        pltpu.VMEM((2, CHUNK, D), jnp.float32),
        pltpu.VMEM((2, 2, CHUNK), jnp.int32),
        pltpu.VMEM((2, LANES, CHUNK), jnp.float32),
        pltpu.VMEM((8, 128), jnp.int32),
        pltpu.VMEM_SHARED((HALF, D), jnp.float32),
        pltpu.SemaphoreType.DMA, pltpu.SemaphoreType.DMA,
    ],
)
def _spmm_sc(h_hbm, zeros_hbm, idx_hbm, w_hbm, cnt_hbm, out_hbm,
             rows_v, idx_v, w_v, cnt_v, acc_sh, sg0, sg1):
    c = lax.axis_index("c")
    s = lax.axis_index("s")
    sg = (sg0, sg1)

    pltpu.sync_copy(cnt_hbm.at[c], cnt_v)
    cpw = cnt_v[0, pl.ds(0, LANES)][0]

    pltpu.sync_copy(zeros_hbm.at[pl.ds(s * RPH, RPH)],
                    acc_sh.at[pl.ds(s * RPH, RPH)])
    plsc.subcore_barrier()

    def meta_copy_sync(slot, t):
        g = s * cpw + t
        pltpu.sync_copy(idx_hbm.at[c, g], idx_v.at[slot])
        pltpu.sync_copy(w_hbm.at[c, g], w_v.at[slot])

    def gather_start(slot):
        pltpu.async_copy(h_hbm.at[idx_v.at[slot, 0]], rows_v.at[slot],
                         sg[slot])

    def gather_wait(slot):
        pltpu.make_async_copy(h_hbm.at[idx_v.at[slot, 0]], rows_v.at[slot],
                              sg[slot]).wait()

    meta_copy_sync(0, 0)
    gather_start(0)

    def step(j, b, prefetch):
        nb = 1 - b
        if prefetch:
            meta_copy_sync(nb, j + 1)
            gather_start(nb)

        gather_wait(b)

        def edge_body(i, carry):
            wv = w_v[b, i // 8, pl.ds((i % 8) * LANES, LANES)]
            for jj in range(D // LANES):
                sl = (b, i, pl.ds(jj * LANES, LANES))
                rows_v[sl] = rows_v[sl] * wv
            return carry
        lax.fori_loop(0, CHUNK, edge_body, 0, unroll=4)

        pltpu.sync_copy(rows_v.at[b], acc_sh.at[idx_v.at[b, 1]], add=True)

    def loop_body(jj, carry):
        step(2 * jj, 0, True)
        step(2 * jj + 1, 1, True)
        return carry
    lax.fori_loop(0, (cpw - 2) // 2, loop_body, 0)
    step(cpw - 2, 0, True)
    step(cpw - 1, 1, False)

    plsc.subcore_barrier()

    pltpu.sync_copy(acc_sh.at[pl.ds(s * RPH, RPH)],
                    out_hbm.at[pl.ds(c * HALF + s * RPH, RPH)])


_BN = 1000


def _linear_tc(h, W, b2):
    def body(h_ref, w_ref, b_ref, o_ref):
        acc = lax.dot_general(h_ref[...], w_ref[...],
                              (((1,), (1,)), ((), ())),
                              preferred_element_type=jnp.float32)
        o_ref[...] = acc + b_ref[...]
    return pl.pallas_call(
        body,
        grid=(N // _BN,),
        in_specs=[
            pl.BlockSpec((_BN, D), lambda i: (i, 0)),
            pl.BlockSpec((D, D), lambda i: (0, 0)),
            pl.BlockSpec((1, D), lambda i: (0, 0)),
        ],
        out_specs=pl.BlockSpec((_BN, D), lambda i: (i, 0)),
        out_shape=jax.ShapeDtypeStruct((N, D), jnp.float32),
    )(h, W, b2)


def kernel(x, edge_index, edge_weight, W, b, k):
    row = edge_index[0]
    col = edge_index[1]

    side = row >= HALF
    sidx = jnp.cumsum(side.astype(jnp.int32))
    n1 = sidx[-1]
    n0 = E - n1
    ar = jnp.arange(E, dtype=jnp.int32)
    pos = jnp.where(side, sidx - 1, ar - sidx)
    tgt = jnp.where(side, CAPE + pos, pos)
    colp = jnp.zeros((2 * CAPE,), jnp.int32).at[tgt].set(col)
    rowp = jnp.zeros((2 * CAPE,), jnp.int32).at[tgt].set(
        row - side.astype(jnp.int32) * HALF)
    wp = jnp.zeros((2 * CAPE,), jnp.float32).at[tgt].set(edge_weight)

    idx = jnp.concatenate(
        [colp.reshape(2, CAPC, 1, CHUNK), rowp.reshape(2, CAPC, 1, CHUNK)],
        axis=2)
    wexp = jnp.broadcast_to(
        wp.reshape(2, CAPC, CHUNK, 1),
        (2, CAPC, CHUNK, LANES)).reshape(2, CAPC, LANES, CHUNK)

    def _cpw(n):
        chunks = (n + CHUNK - 1) // CHUNK
        return jnp.maximum(4 * ((chunks + 4 * NS - 1) // (4 * NS)), 4)
    cnts = jnp.broadcast_to(
        jnp.stack([_cpw(n0), _cpw(n1)]).astype(jnp.int32)[:, None, None],
        (2, 8, 128))

    zeros = jnp.zeros((HALF, D), jnp.float32)
    b2 = b.reshape(1, D)
    x_pad = jnp.pad(x, ((0, N_PAD - N), (0, 0)))

    def it_body(_, h):
        return _spmm_sc(h, zeros, idx, wexp, cnts)

    h = lax.fori_loop(0, k, it_body, x_pad)
    return _linear_tc(h[:N], W, b2)


# speedup vs baseline: 1.0681x; 1.0001x over previous
import functools

import jax
import jax.numpy as jnp
from jax import lax
from jax.experimental import pallas as pl
from jax.experimental.pallas import tpu as pltpu
from jax.experimental.pallas import tpu_sc as plsc

N = 10000
E = 320000
D = 128

NC = 2
NS = 16
LANES = 16

CHUNK = 128
HALF = 5120
N_PAD = 2 * HALF
RPH = HALF // NS
CAPC = 2560
CAPE = CAPC * CHUNK

_mesh = plsc.VectorSubcoreMesh(
    core_axis_name="c", subcore_axis_name="s", num_cores=NC, num_subcores=NS)


@functools.partial(
    pl.kernel,
    out_type=jax.ShapeDtypeStruct((N_PAD, D), jnp.float32),
    mesh=_mesh,
    scratch_types=[
        pltpu.VMEM((2, CHUNK, D), jnp.float32),
        pltpu.VMEM((2, 2, CHUNK), jnp.int32),
        pltpu.VMEM((2, LANES, CHUNK), jnp.float32),
        pltpu.VMEM((8, 128), jnp.int32),
        pltpu.VMEM_SHARED((HALF, D), jnp.float32),
        pltpu.SemaphoreType.DMA, pltpu.SemaphoreType.DMA,
    ],
)
def _spmm_sc(h_hbm, zeros_hbm, idx_hbm, w_hbm, cnt_hbm, out_hbm,
             rows_v, idx_v, w_v, cnt_v, acc_sh, sg0, sg1):
    c = lax.axis_index("c")
    s = lax.axis_index("s")
    sg = (sg0, sg1)

    pltpu.sync_copy(cnt_hbm.at[c], cnt_v)
    cpw = cnt_v[0, pl.ds(0, LANES)][0]

    pltpu.sync_copy(zeros_hbm.at[pl.ds(s * RPH, RPH)],
                    acc_sh.at[pl.ds(s * RPH, RPH)])
    plsc.subcore_barrier()

    def meta_copy_sync(slot, t):
        g = s * cpw + t
        pltpu.sync_copy(idx_hbm.at[c, g], idx_v.at[slot])
        pltpu.sync_copy(w_hbm.at[c, g], w_v.at[slot])

    def gather_start(slot):
        pltpu.async_copy(h_hbm.at[idx_v.at[slot, 0]], rows_v.at[slot],
                         sg[slot])

    def gather_wait(slot):
        pltpu.make_async_copy(h_hbm.at[idx_v.at[slot, 0]], rows_v.at[slot],
                              sg[slot]).wait()

    meta_copy_sync(0, 0)
    gather_start(0)
    _ns = jax.named_scope

    def step(j, b, prefetch):
        nb = 1 - b
        if prefetch:
            meta_copy_sync(nb, j + 1)
            gather_start(nb)

        gather_wait(b)

        def edge_body(i, carry):
            wv = w_v[b, i // 8, pl.ds((i % 8) * LANES, LANES)]
            for jj in range(D // LANES):
                sl = (b, i, pl.ds(jj * LANES, LANES))
                rows_v[sl] = rows_v[sl] * wv
            return carry
        lax.fori_loop(0, CHUNK, edge_body, 0, unroll=4)

        pltpu.sync_copy(rows_v.at[b], acc_sh.at[idx_v.at[b, 1]], add=True)

    def loop_body(jj, carry):
        step(2 * jj, 0, True)
        step(2 * jj + 1, 1, True)
        return carry
    with _ns("mainloop"):
        lax.fori_loop(0, (cpw - 2) // 2, loop_body, 0)
    with _ns("tail"):
        step(cpw - 2, 0, True)
        step(cpw - 1, 1, False)

    plsc.subcore_barrier()

    pltpu.sync_copy(acc_sh.at[pl.ds(s * RPH, RPH)],
                    out_hbm.at[pl.ds(c * HALF + s * RPH, RPH)])


_BN = 1000


def _linear_tc(h, W, b2):
    def body(h_ref, w_ref, b_ref, o_ref):
        acc = lax.dot_general(h_ref[...], w_ref[...],
                              (((1,), (1,)), ((), ())),
                              preferred_element_type=jnp.float32)
        o_ref[...] = acc + b_ref[...]
    return pl.pallas_call(
        body,
        grid=(N // _BN,),
        in_specs=[
            pl.BlockSpec((_BN, D), lambda i: (i, 0)),
            pl.BlockSpec((D, D), lambda i: (0, 0)),
            pl.BlockSpec((1, D), lambda i: (0, 0)),
        ],
        out_specs=pl.BlockSpec((_BN, D), lambda i: (i, 0)),
        out_shape=jax.ShapeDtypeStruct((N, D), jnp.float32),
    )(h, W, b2)


def kernel(x, edge_index, edge_weight, W, b, k):
    row = edge_index[0]
    col = edge_index[1]

    side = row >= HALF
    sidx = jnp.cumsum(side.astype(jnp.int32))
    n1 = sidx[-1]
    n0 = E - n1
    ar = jnp.arange(E, dtype=jnp.int32)
    pos = jnp.where(side, sidx - 1, ar - sidx)
    tgt = jnp.where(side, CAPE + pos, pos)
    colp = jnp.zeros((2 * CAPE,), jnp.int32).at[tgt].set(col)
    rowp = jnp.zeros((2 * CAPE,), jnp.int32).at[tgt].set(
        row - side.astype(jnp.int32) * HALF)
    wp = jnp.zeros((2 * CAPE,), jnp.float32).at[tgt].set(edge_weight)

    idx = jnp.concatenate(
        [colp.reshape(2, CAPC, 1, CHUNK), rowp.reshape(2, CAPC, 1, CHUNK)],
        axis=2)
    wexp = jnp.broadcast_to(
        wp.reshape(2, CAPC, CHUNK, 1),
        (2, CAPC, CHUNK, LANES)).reshape(2, CAPC, LANES, CHUNK)

    def _cpw(n):
        chunks = (n + CHUNK - 1) // CHUNK
        return jnp.maximum(4 * ((chunks + 4 * NS - 1) // (4 * NS)), 4)
    cnts = jnp.broadcast_to(
        jnp.stack([_cpw(n0), _cpw(n1)]).astype(jnp.int32)[:, None, None],
        (2, 8, 128))

    zeros = jnp.zeros((HALF, D), jnp.float32)
    b2 = b.reshape(1, D)
    x_pad = jnp.pad(x, ((0, N_PAD - N), (0, 0)))

    def it_body(_, h):
        return _spmm_sc(h, zeros, idx, wexp, cnts)

    h = lax.fori_loop(0, k, it_body, x_pad)
    return _linear_tc(h[:N], W, b2)


# speedup vs baseline: 3.9840x; 3.7299x over previous
import functools

import jax
import jax.numpy as jnp
from jax import lax
from jax.experimental import pallas as pl
from jax.experimental.pallas import tpu as pltpu
from jax.experimental.pallas import tpu_sc as plsc

N = 10000
E = 320000
D = 128

NC = 2
NS = 16
NW = NC * NS
LANES = 16

CHUNK = 128
CHUNKS_PAD = 2560
E_PAD = CHUNKS_PAD * CHUNK
CPW0 = 98
CPW1 = 62
NCH0 = NS * CPW0
RPT = 8 * (-(-N // (8 * NS)))
N_PAD = RPT * NS

_mesh = plsc.VectorSubcoreMesh(
    core_axis_name="c", subcore_axis_name="s", num_cores=NC, num_subcores=NS)


@functools.partial(
    pl.kernel,
    out_type=jax.ShapeDtypeStruct((NC, N_PAD, D), jnp.float32),
    mesh=_mesh,
    scratch_types=[
        pltpu.VMEM((2, CHUNK, D), jnp.float32),
        pltpu.VMEM((2, 2, CHUNK), jnp.int32),
        pltpu.VMEM((2, LANES, CHUNK), jnp.float32),
        pltpu.VMEM_SHARED((N_PAD, D), jnp.float32),
        pltpu.SemaphoreType.DMA,
        pltpu.SemaphoreType.DMA,
    ],
)
def _spmm_sc(h_hbm, zeros_hbm, idx_hbm, w_hbm, out_hbm,
             rows_v, idx_v, w_v, acc_sh, sg0, sg1):
    c = lax.axis_index("c")
    s = lax.axis_index("s")
    sg = (sg0, sg1)

    pltpu.sync_copy(zeros_hbm.at[pl.ds(s * RPT, RPT)],
                    acc_sh.at[pl.ds(s * RPT, RPT)])
    plsc.subcore_barrier()

    def run(cpw, base):
        def meta_copy_sync(slot, j):
            pltpu.sync_copy(idx_hbm.at[base + j], idx_v.at[slot])
            pltpu.sync_copy(w_hbm.at[base + j], w_v.at[slot])

        def gather_start(slot):
            pltpu.async_copy(h_hbm.at[idx_v.at[slot, 0]], rows_v.at[slot],
                             sg[slot])

        def gather_wait(slot):
            pltpu.make_async_copy(h_hbm.at[idx_v.at[slot, 0]],
                                  rows_v.at[slot], sg[slot]).wait()

        meta_copy_sync(0, 0)
        gather_start(0)

        def step(j, b):
            nb = 1 - b
            @pl.when(j + 1 < cpw)
            def _():
                meta_copy_sync(nb, j + 1)
                gather_start(nb)

            gather_wait(b)

            def edge_body(i, carry):
                wv = w_v[b, i // 8, pl.ds((i % 8) * LANES, LANES)]
                for jj in range(D // LANES):
                    sl = (b, i, pl.ds(jj * LANES, LANES))
                    rows_v[sl] = rows_v[sl] * wv
                return carry
            lax.fori_loop(0, CHUNK, edge_body, 0, unroll=4)

            pltpu.sync_copy(rows_v.at[b], acc_sh.at[idx_v.at[b, 1]],
                            add=True)

        def loop_body(jj, carry):
            step(2 * jj, 0)
            step(2 * jj + 1, 1)
            return carry
        lax.fori_loop(0, cpw // 2, loop_body, 0)

    @pl.when(c == 0)
    def _():
        run(CPW0, s * CPW0)

    @pl.when(c == 1)
    def _():
        run(CPW1, NCH0 + s * CPW1)

    plsc.subcore_barrier()
    pltpu.sync_copy(acc_sh.at[pl.ds(s * RPT, RPT)],
                    out_hbm.at[c, pl.ds(s * RPT, RPT)])


_BN = 1000
_BC = RPT


def _combine_tc(p):
    def body(p_ref, o_ref):
        o_ref[...] = p_ref[0] + p_ref[1]
    return pl.pallas_call(
        body,
        grid=(N_PAD // _BC,),
        in_specs=[pl.BlockSpec((2, _BC, D), lambda i: (0, i, 0))],
        out_specs=pl.BlockSpec((_BC, D), lambda i: (i, 0)),
        out_shape=jax.ShapeDtypeStruct((N_PAD, D), jnp.float32),
    )(p)


def _linear_tc(h, W, b2):
    def body(h_ref, w_ref, b_ref, o_ref):
        acc = lax.dot_general(h_ref[...], w_ref[...],
                              (((1,), (1,)), ((), ())),
                              preferred_element_type=jnp.float32)
        o_ref[...] = acc + b_ref[...]
    return pl.pallas_call(
        body,
        grid=(N // _BN,),
        in_specs=[
            pl.BlockSpec((_BN, D), lambda i: (i, 0)),
            pl.BlockSpec((D, D), lambda i: (0, 0)),
            pl.BlockSpec((1, D), lambda i: (0, 0)),
        ],
        out_specs=pl.BlockSpec((_BN, D), lambda i: (i, 0)),
        out_shape=jax.ShapeDtypeStruct((N, D), jnp.float32),
    )(h, W, b2)


def kernel(x, edge_index, edge_weight, W, b, k):
    row = edge_index[0]
    col = edge_index[1]
    pad = E_PAD - E
    col2 = jnp.pad(col, (0, pad)).reshape(CHUNKS_PAD, 1, CHUNK)
    row2 = jnp.pad(row, (0, pad)).reshape(CHUNKS_PAD, 1, CHUNK)
    idx = jnp.concatenate([col2, row2], axis=1)
    w2 = jnp.broadcast_to(
        jnp.pad(edge_weight, (0, pad)).reshape(CHUNKS_PAD, CHUNK, 1),
        (CHUNKS_PAD, CHUNK, LANES)).reshape(CHUNKS_PAD, LANES, CHUNK)
    zeros = jnp.zeros((N_PAD, D), jnp.float32)
    b2 = b.reshape(1, D)
    x_pad = jnp.pad(x, ((0, N_PAD - N), (0, 0)))

    def it_body(_, h):
        p = _spmm_sc(h, zeros, idx, w2)
        return _combine_tc(p)

    h = lax.fori_loop(0, k, it_body, x_pad)
    return _linear_tc(h[:N], W, b2)


# speedup vs baseline: 4.6550x; 1.1684x over previous
import functools

import jax
import jax.numpy as jnp
from jax import lax
from jax.experimental import pallas as pl
from jax.experimental.pallas import tpu as pltpu
from jax.experimental.pallas import tpu_sc as plsc

N = 10000
E = 320000
D = 128

NC = 2
NS = 16
NW = NC * NS
LANES = 16

CHUNK = 128
CHUNKS_PAD = 2560
E_PAD = CHUNKS_PAD * CHUNK
CPW0 = 106
CPW1 = 54
NCH0 = NS * CPW0
RPT = 8 * (-(-N // (8 * NS)))
N_PAD = RPT * NS

_mesh = plsc.VectorSubcoreMesh(
    core_axis_name="c", subcore_axis_name="s", num_cores=NC, num_subcores=NS)


@functools.partial(
    pl.kernel,
    out_type=jax.ShapeDtypeStruct((NC, N_PAD, D), jnp.float32),
    mesh=_mesh,
    scratch_types=[
        pltpu.VMEM((2, CHUNK, D), jnp.float32),
        pltpu.VMEM((2, 2, CHUNK), jnp.int32),
        pltpu.VMEM((2, LANES, CHUNK), jnp.float32),
        pltpu.VMEM_SHARED((N_PAD, D), jnp.float32),
        pltpu.SemaphoreType.DMA,
        pltpu.SemaphoreType.DMA,
    ],
)
def _spmm_sc(h_hbm, zeros_hbm, idx_hbm, w_hbm, out_hbm,
             rows_v, idx_v, w_v, acc_sh, sg0, sg1):
    c = lax.axis_index("c")
    s = lax.axis_index("s")
    sg = (sg0, sg1)

    pltpu.sync_copy(zeros_hbm.at[pl.ds(s * RPT, RPT)],
                    acc_sh.at[pl.ds(s * RPT, RPT)])
    plsc.subcore_barrier()

    def run(cpw, base):
        def meta_copy_sync(slot, j):
            pltpu.sync_copy(idx_hbm.at[base + j], idx_v.at[slot])
            pltpu.sync_copy(w_hbm.at[base + j], w_v.at[slot])

        def gather_start(slot):
            pltpu.async_copy(h_hbm.at[idx_v.at[slot, 0]], rows_v.at[slot],
                             sg[slot])

        def gather_wait(slot):
            pltpu.make_async_copy(h_hbm.at[idx_v.at[slot, 0]],
                                  rows_v.at[slot], sg[slot]).wait()

        meta_copy_sync(0, 0)
        gather_start(0)

        def step(j, b):
            nb = 1 - b
            @pl.when(j + 1 < cpw)
            def _():
                meta_copy_sync(nb, j + 1)
                gather_start(nb)

            gather_wait(b)

            def edge_body(i, carry):
                wv = w_v[b, i // 8, pl.ds((i % 8) * LANES, LANES)]
                for jj in range(D // LANES):
                    sl = (b, i, pl.ds(jj * LANES, LANES))
                    rows_v[sl] = rows_v[sl] * wv
                return carry
            lax.fori_loop(0, CHUNK, edge_body, 0, unroll=4)

            pltpu.sync_copy(rows_v.at[b], acc_sh.at[idx_v.at[b, 1]],
                            add=True)

        def loop_body(jj, carry):
            step(2 * jj, 0)
            step(2 * jj + 1, 1)
            return carry
        lax.fori_loop(0, cpw // 2, loop_body, 0)

    @pl.when(c == 0)
    def _():
        run(CPW0, s * CPW0)

    @pl.when(c == 1)
    def _():
        run(CPW1, NCH0 + s * CPW1)

    plsc.subcore_barrier()
    pltpu.sync_copy(acc_sh.at[pl.ds(s * RPT, RPT)],
                    out_hbm.at[c, pl.ds(s * RPT, RPT)])


_BN = 1000
_BC = RPT


def _combine_tc(p):
    def body(p_ref, o_ref):
        o_ref[...] = p_ref[0] + p_ref[1]
    return pl.pallas_call(
        body,
        grid=(N_PAD // _BC,),
        in_specs=[pl.BlockSpec((2, _BC, D), lambda i: (0, i, 0))],
        out_specs=pl.BlockSpec((_BC, D), lambda i: (i, 0)),
        out_shape=jax.ShapeDtypeStruct((N_PAD, D), jnp.float32),
    )(p)


def _linear_tc(h, W, b2):
    def body(h_ref, w_ref, b_ref, o_ref):
        acc = lax.dot_general(h_ref[...], w_ref[...],
                              (((1,), (1,)), ((), ())),
                              preferred_element_type=jnp.float32)
        o_ref[...] = acc + b_ref[...]
    return pl.pallas_call(
        body,
        grid=(N // _BN,),
        in_specs=[
            pl.BlockSpec((_BN, D), lambda i: (i, 0)),
            pl.BlockSpec((D, D), lambda i: (0, 0)),
            pl.BlockSpec((1, D), lambda i: (0, 0)),
        ],
        out_specs=pl.BlockSpec((_BN, D), lambda i: (i, 0)),
        out_shape=jax.ShapeDtypeStruct((N, D), jnp.float32),
    )(h, W, b2)


def kernel(x, edge_index, edge_weight, W, b, k):
    row = edge_index[0]
    col = edge_index[1]
    pad = E_PAD - E
    spread = (jnp.arange(pad, dtype=jnp.int32) * 37) % N
    col2 = jnp.concatenate([col, spread]).reshape(CHUNKS_PAD, 1, CHUNK)
    row2 = jnp.concatenate([row, spread]).reshape(CHUNKS_PAD, 1, CHUNK)
    idx = jnp.concatenate([col2, row2], axis=1)
    w2 = jnp.broadcast_to(
        jnp.pad(edge_weight, (0, pad)).reshape(CHUNKS_PAD, CHUNK, 1),
        (CHUNKS_PAD, CHUNK, LANES)).reshape(CHUNKS_PAD, LANES, CHUNK)
    zeros = jnp.zeros((N_PAD, D), jnp.float32)
    b2 = b.reshape(1, D)
    x_pad = jnp.pad(x, ((0, N_PAD - N), (0, 0)))

    def it_body(_, h):
        p = _spmm_sc(h, zeros, idx, w2)
        return _combine_tc(p)

    h = lax.fori_loop(0, k, it_body, x_pad)
    return _linear_tc(h[:N], W, b2)


# speedup vs baseline: 5.7839x; 1.2425x over previous
import functools

import jax
import jax.numpy as jnp
from jax import lax
from jax.experimental import pallas as pl
from jax.experimental.pallas import tpu as pltpu
from jax.experimental.pallas import tpu_sc as plsc

N = 10000
E = 320000
D = 128

NC = 2
NS = 16
NW = NC * NS
LANES = 16

CHUNK = 128
CHUNKS_PAD = 2560
E_PAD = CHUNKS_PAD * CHUNK
CPW0 = 80
CPW1 = 80
NCH0 = NS * CPW0
RPT = 8 * (-(-N // (8 * NS)))
N_PAD = RPT * NS

_mesh = plsc.VectorSubcoreMesh(
    core_axis_name="c", subcore_axis_name="s", num_cores=NC, num_subcores=NS)


@functools.partial(
    pl.kernel,
    out_type=jax.ShapeDtypeStruct((NC, N_PAD, D), jnp.float32),
    mesh=_mesh,
    scratch_types=[
        pltpu.VMEM((2, CHUNK, D), jnp.float32),
        pltpu.VMEM((2, 2, CHUNK), jnp.int32),
        pltpu.VMEM((2, LANES, CHUNK), jnp.float32),
        pltpu.VMEM_SHARED((N_PAD, D), jnp.float32),
        pltpu.SemaphoreType.DMA,
        pltpu.SemaphoreType.DMA,
    ],
)
def _spmm_sc(h_hbm, zeros_hbm, idx_hbm, w_hbm, out_hbm,
             rows_v, idx_v, w_v, acc_sh, sg0, sg1):
    c = lax.axis_index("c")
    s = lax.axis_index("s")
    sg = (sg0, sg1)

    pltpu.sync_copy(zeros_hbm.at[pl.ds(s * RPT, RPT)],
                    acc_sh.at[pl.ds(s * RPT, RPT)])
    plsc.subcore_barrier()

    def run(cpw, base):
        def meta_copy_sync(slot, j):
            pltpu.sync_copy(idx_hbm.at[base + j], idx_v.at[slot])
            pltpu.sync_copy(w_hbm.at[base + j], w_v.at[slot])

        def gather_start(slot):
            pltpu.async_copy(h_hbm.at[idx_v.at[slot, 0]], rows_v.at[slot],
                             sg[slot])

        def gather_wait(slot):
            pltpu.make_async_copy(h_hbm.at[idx_v.at[slot, 0]],
                                  rows_v.at[slot], sg[slot]).wait()

        meta_copy_sync(0, 0)
        gather_start(0)

        def step(j, b):
            nb = 1 - b
            @pl.when(j + 1 < cpw)
            def _():
                meta_copy_sync(nb, j + 1)
                gather_start(nb)

            gather_wait(b)

            def edge_body(i, carry):
                wv = w_v[b, i // 8, pl.ds((i % 8) * LANES, LANES)]
                for jj in range(D // LANES):
                    sl = (b, i, pl.ds(jj * LANES, LANES))
                    rows_v[sl] = rows_v[sl] * wv
                return carry
            lax.fori_loop(0, CHUNK, edge_body, 0, unroll=4)

            pltpu.sync_copy(rows_v.at[b], acc_sh.at[idx_v.at[b, 1]],
                            add=True)

        def loop_body(jj, carry):
            step(2 * jj, 0)
            step(2 * jj + 1, 1)
            return carry
        lax.fori_loop(0, cpw // 2, loop_body, 0)

    @pl.when(c == 0)
    def _():
        run(CPW0, s * CPW0)

    @pl.when(c == 1)
    def _():
        run(CPW1, NCH0 + s * CPW1)

    plsc.subcore_barrier()
    pltpu.sync_copy(acc_sh.at[pl.ds(s * RPT, RPT)],
                    out_hbm.at[c, pl.ds(s * RPT, RPT)])


_BN = 1000
_BC = RPT


def _combine_tc(p):
    def body(p_ref, o_ref):
        o_ref[...] = p_ref[0] + p_ref[1]
    return pl.pallas_call(
        body,
        grid=(N_PAD // _BC,),
        in_specs=[pl.BlockSpec((2, _BC, D), lambda i: (0, i, 0))],
        out_specs=pl.BlockSpec((_BC, D), lambda i: (i, 0)),
        out_shape=jax.ShapeDtypeStruct((N_PAD, D), jnp.float32),
    )(p)


def _linear_tc(h, W, b2):
    def body(h_ref, w_ref, b_ref, o_ref):
        acc = lax.dot_general(h_ref[...], w_ref[...],
                              (((1,), (1,)), ((), ())),
                              preferred_element_type=jnp.float32)
        o_ref[...] = acc + b_ref[...]
    return pl.pallas_call(
        body,
        grid=(N // _BN,),
        in_specs=[
            pl.BlockSpec((_BN, D), lambda i: (i, 0)),
            pl.BlockSpec((D, D), lambda i: (0, 0)),
            pl.BlockSpec((1, D), lambda i: (0, 0)),
        ],
        out_specs=pl.BlockSpec((_BN, D), lambda i: (i, 0)),
        out_shape=jax.ShapeDtypeStruct((N, D), jnp.float32),
    )(h, W, b2)


def kernel(x, edge_index, edge_weight, W, b, k):
    row = edge_index[0]
    col = edge_index[1]
    pad = E_PAD - E
    spread = (jnp.arange(pad, dtype=jnp.int32) * 37) % N
    col2 = jnp.concatenate([col, spread]).reshape(CHUNKS_PAD, 1, CHUNK)
    row2 = jnp.concatenate([row, spread]).reshape(CHUNKS_PAD, 1, CHUNK)
    idx = jnp.concatenate([col2, row2], axis=1)
    w2 = jnp.broadcast_to(
        jnp.pad(edge_weight, (0, pad)).reshape(CHUNKS_PAD, CHUNK, 1),
        (CHUNKS_PAD, CHUNK, LANES)).reshape(CHUNKS_PAD, LANES, CHUNK)
    zeros = jnp.zeros((N_PAD, D), jnp.float32)
    b2 = b.reshape(1, D)
    x_pad = jnp.pad(x, ((0, N_PAD - N), (0, 0)))

    def it_body(_, h):
        p = _spmm_sc(h, zeros, idx, w2)
        return _combine_tc(p)

    h = lax.fori_loop(0, k, it_body, x_pad)
    return _linear_tc(h[:N], W, b2)


# speedup vs baseline: 7.9184x; 1.3690x over previous
import functools

import jax
import jax.numpy as jnp
from jax import lax
from jax.experimental import pallas as pl
from jax.experimental.pallas import tpu as pltpu
from jax.experimental.pallas import tpu_sc as plsc

N = 10000
E = 320000
D = 128

NC = 2
NS = 16
NW = NC * NS
LANES = 16

CHUNK = 128
CPW = 84
CHUNKS_PAD = CPW * NW
E_PAD = CHUNKS_PAD * CHUNK
RPT = 8 * (-(-N // (8 * NS)))
N_PAD = RPT * NS

_mesh = plsc.VectorSubcoreMesh(
    core_axis_name="c", subcore_axis_name="s", num_cores=NC, num_subcores=NS)


@functools.partial(
    pl.kernel,
    out_type=jax.ShapeDtypeStruct((NC, N_PAD, D), jnp.float32),
    mesh=_mesh,
    scratch_types=[
        pltpu.VMEM((2, CHUNK, D), jnp.float32),
        pltpu.VMEM((3, 2, CHUNK), jnp.int32),
        pltpu.VMEM((3, LANES, CHUNK), jnp.float32),
        pltpu.VMEM_SHARED((N_PAD, D), jnp.float32),
        pltpu.SemaphoreType.DMA, pltpu.SemaphoreType.DMA,
        pltpu.SemaphoreType.DMA, pltpu.SemaphoreType.DMA,
        pltpu.SemaphoreType.DMA,
    ],
)
def _spmm_sc(h_hbm, zeros_hbm, idx_hbm, w_hbm, out_hbm,
             rows_v, idx_v, w_v, acc_sh, sg0, sg1, si0, si1, si2):
    c = lax.axis_index("c")
    s = lax.axis_index("s")
    sg = (sg0, sg1)
    si = (si0, si1, si2)

    pltpu.sync_copy(zeros_hbm.at[pl.ds(s * RPT, RPT)],
                    acc_sh.at[pl.ds(s * RPT, RPT)])
    plsc.subcore_barrier()

    base = (c * NS + s) * CPW

    def meta_copy(m, j):
        pltpu.async_copy(idx_hbm.at[base + j], idx_v.at[m], si[m])
        pltpu.async_copy(w_hbm.at[base + j], w_v.at[m], si[m])

    def meta_wait(m, j):
        pltpu.make_async_copy(idx_hbm.at[base + j], idx_v.at[m],
                              si[m]).wait()
        pltpu.make_async_copy(w_hbm.at[base + j], w_v.at[m], si[m]).wait()

    def gather_start(b, m):
        pltpu.async_copy(h_hbm.at[idx_v.at[m, 0]], rows_v.at[b], sg[b])

    def gather_wait(b, m):
        pltpu.make_async_copy(h_hbm.at[idx_v.at[m, 0]], rows_v.at[b],
                              sg[b]).wait()

    meta_copy(0, 0)
    meta_wait(0, 0)
    meta_copy(1, 1)
    gather_start(0, 0)

    def step(j, t, pf_gather, pf_meta):
        b = t % 2
        m = t % 3
        if pf_gather:
            meta_wait((t + 1) % 3, j + 1)
            gather_start(1 - b, (t + 1) % 3)
        if pf_meta:
            meta_copy((t + 2) % 3, j + 2)

        gather_wait(b, m)

        def edge_body(i, carry):
            wv = w_v[m, i // 8, pl.ds((i % 8) * LANES, LANES)]
            for jj in range(D // LANES):
                sl = (b, i, pl.ds(jj * LANES, LANES))
                rows_v[sl] = rows_v[sl] * wv
            return carry
        lax.fori_loop(0, CHUNK, edge_body, 0, unroll=4)

        pltpu.sync_copy(rows_v.at[b], acc_sh.at[idx_v.at[m, 1]], add=True)

    def loop_body(jj, carry):
        for t in range(6):
            step(6 * jj + t, t, True, True)
        return carry
    lax.fori_loop(0, CPW // 6 - 1, loop_body, 0)
    for t in range(6):
        j = CPW - 6 + t
        step(j, t, j + 1 < CPW, j + 2 < CPW)

    plsc.subcore_barrier()
    pltpu.sync_copy(acc_sh.at[pl.ds(s * RPT, RPT)],
                    out_hbm.at[c, pl.ds(s * RPT, RPT)])


_BN = 1000
_BC = RPT


def _combine_tc(p):
    def body(p_ref, o_ref):
        o_ref[...] = p_ref[0] + p_ref[1]
    return pl.pallas_call(
        body,
        grid=(N_PAD // _BC,),
        in_specs=[pl.BlockSpec((2, _BC, D), lambda i: (0, i, 0))],
        out_specs=pl.BlockSpec((_BC, D), lambda i: (i, 0)),
        out_shape=jax.ShapeDtypeStruct((N_PAD, D), jnp.float32),
    )(p)


def _linear_tc(h, W, b2):
    def body(h_ref, w_ref, b_ref, o_ref):
        acc = lax.dot_general(h_ref[...], w_ref[...],
                              (((1,), (1,)), ((), ())),
                              preferred_element_type=jnp.float32)
        o_ref[...] = acc + b_ref[...]
    return pl.pallas_call(
        body,
        grid=(N // _BN,),
        in_specs=[
            pl.BlockSpec((_BN, D), lambda i: (i, 0)),
            pl.BlockSpec((D, D), lambda i: (0, 0)),
            pl.BlockSpec((1, D), lambda i: (0, 0)),
        ],
        out_specs=pl.BlockSpec((_BN, D), lambda i: (i, 0)),
        out_shape=jax.ShapeDtypeStruct((N, D), jnp.float32),
    )(h, W, b2)


def kernel(x, edge_index, edge_weight, W, b, k):
    row = edge_index[0]
    col = edge_index[1]
    pad = E_PAD - E
    spread = (jnp.arange(pad, dtype=jnp.int32) * 37) % N
    col2 = jnp.concatenate([col, spread]).reshape(CHUNKS_PAD, 1, CHUNK)
    row2 = jnp.concatenate([row, spread]).reshape(CHUNKS_PAD, 1, CHUNK)
    idx = jnp.concatenate([col2, row2], axis=1)
    w2 = jnp.broadcast_to(
        jnp.pad(edge_weight, (0, pad)).reshape(CHUNKS_PAD, CHUNK, 1),
        (CHUNKS_PAD, CHUNK, LANES)).reshape(CHUNKS_PAD, LANES, CHUNK)
    zeros = jnp.zeros((N_PAD, D), jnp.float32)
    b2 = b.reshape(1, D)
    x_pad = jnp.pad(x, ((0, N_PAD - N), (0, 0)))

    def it_body(_, h):
        p = _spmm_sc(h, zeros, idx, w2)
        return _combine_tc(p)

    h = lax.fori_loop(0, k, it_body, x_pad)
    return _linear_tc(h[:N], W, b2)
